# baseline (speedup 1.0000x reference)
import functools

import jax
import jax.numpy as jnp
import numpy as np
from jax import lax
from jax.experimental import pallas as pl
from jax.experimental.pallas import tpu as pltpu
from jax.experimental.pallas import tpu_sc as plsc

N_NODES = 10000
N_EDGES = 320000
D = 128
H = D // 2

NC = 2
NS = 16

CHUNK = 80
CHUNKS_PER_TILE = N_EDGES // (NS * CHUNK)

NP = 10240
ROWS_PER_TILE = NP // NS

BM = 400

_PERM = np.empty(D, dtype=np.int32)
for _h in range(4):
    _base = 32 * _h
    for _k in range(16):
        _PERM[_base + 2 * _k] = _base + _k
        _PERM[_base + 2 * _k + 1] = _base + 16 + _k


def _mm_body(x_ref, w1_ref, w2_ref, b_ref, t_ref):
    xb = x_ref[...]
    m1 = jnp.dot(xb, w1_ref[...], preferred_element_type=jnp.float32) + b_ref[...]
    m2 = jnp.dot(xb, w2_ref[...], preferred_element_type=jnp.float32)
    t_ref[0, 0] = m2[:, :H].astype(jnp.bfloat16)
    t_ref[0, 1] = m1[:, :H].astype(jnp.bfloat16)
    t_ref[1, 0] = m2[:, H:].astype(jnp.bfloat16)
    t_ref[1, 1] = m1[:, H:].astype(jnp.bfloat16)


def _precompute_table(x, w1, w2, b2d):
    return pl.pallas_call(
        _mm_body,
        grid=(N_NODES // BM,),
        in_specs=[
            pl.BlockSpec((BM, D), lambda i: (i, 0)),
            pl.BlockSpec((D, D), lambda i: (0, 0)),
            pl.BlockSpec((D, D), lambda i: (0, 0)),
            pl.BlockSpec((1, D), lambda i: (0, 0)),
        ],
        out_specs=pl.BlockSpec((NC, 2, BM, H), lambda i: (0, 0, i, 0)),
        out_shape=jax.ShapeDtypeStruct((NC, 2, N_NODES, H), jnp.bfloat16),
    )(x, w1, w2, b2d)


@functools.partial(
    pl.kernel,
    out_type=jax.ShapeDtypeStruct((N_NODES, NC, H), jnp.float32),
    mesh=plsc.VectorSubcoreMesh(core_axis_name="c", subcore_axis_name="s"),
    scratch_types=[
        pltpu.VMEM((CHUNKS_PER_TILE, CHUNK), jnp.int32),
        pltpu.VMEM((CHUNKS_PER_TILE, CHUNK), jnp.int32),
        pltpu.VMEM((CHUNK, H), jnp.bfloat16),
        pltpu.VMEM((CHUNK, H), jnp.bfloat16),
        pltpu.VMEM((CHUNK, H), jnp.bfloat16),
        pltpu.VMEM((CHUNK, H), jnp.bfloat16),
        pltpu.VMEM((CHUNK, H), jnp.float32),
        pltpu.VMEM((CHUNK, H), jnp.float32),
        pltpu.VMEM_SHARED((NP, H), jnp.float32),
        pltpu.SemaphoreType.DMA,
        pltpu.SemaphoreType.DMA,
        pltpu.SemaphoreType.DMA,
        pltpu.SemaphoreType.DMA,
        pltpu.SemaphoreType.DMA,
        pltpu.SemaphoreType.DMA,
    ],
    compiler_params=pltpu.CompilerParams(
        use_tc_tiling_on_sc=False, needs_layout_passes=False
    ),
)
def _sc_edges(t_hbm, srcg_hbm, dstg_hbm, out_hbm,
              sidx, didx, ra0, rb0, ra1, rb1, msg0, msg1, accum,
              sem_a0, sem_b0, sem_a1, sem_b1, sem_s0, sem_s1):
    c = lax.axis_index("c")
    s = lax.axis_index("s")
    tab = t_hbm.at[c]

    cp_si = pltpu.async_copy(srcg_hbm.at[s], sidx, sem_a0)
    cp_di = pltpu.async_copy(dstg_hbm.at[s], didx, sem_b0)

    zero = jnp.zeros((16,), jnp.float32)

    @pl.loop(0, CHUNK)
    def _zero_rows(e):
        for j in range(H // 16):
            msg0[e, pl.ds(j * 16, 16)] = zero
            msg1[e, pl.ds(j * 16, 16)] = zero

    row0 = s * ROWS_PER_TILE

    @pl.loop(0, ROWS_PER_TILE // CHUNK)
    def _zero_accum(k):
        pltpu.sync_copy(msg0, accum.at[pl.ds(row0 + k * CHUNK, CHUNK)])

    cp_si.wait()
    cp_di.wait()

    plsc.subcore_barrier()

    himask = jnp.full((16,), -65536, jnp.int32)

    def _relu_unpack(ra, rb, msg):
        @pl.loop(0, CHUNK, unroll=8)
        def _row(e):
            for j in range(H // 32):
                a32 = plsc.bitcast(ra[e, pl.ds(j * 32, 32)], jnp.int32)
                b32 = plsc.bitcast(rb[e, pl.ds(j * 32, 32)], jnp.int32)
                ae = plsc.bitcast(a32 << 16, jnp.float32)
                be = plsc.bitcast(b32 << 16, jnp.float32)
                ao = plsc.bitcast(a32 & himask, jnp.float32)
                bo = plsc.bitcast(b32 & himask, jnp.float32)
                msg[e, pl.ds(j * 32, 16)] = jnp.maximum(ae + be, 0.0)
                msg[e, pl.ds(j * 32 + 16, 16)] = jnp.maximum(ao + bo, 0.0)

    pltpu.async_copy(tab.at[sidx.at[0]], ra0, sem_a0)
    pltpu.async_copy(tab.at[didx.at[0]], rb0, sem_b0)
    pltpu.async_copy(msg0, accum.at[didx.at[0]], sem_s0, add=True)
    pltpu.async_copy(msg1, accum.at[didx.at[0]], sem_s1, add=True)

    @pl.loop(0, CHUNKS_PER_TILE // 2)
    def _pair(p):
        c0 = 2 * p
        c1 = c0 + 1
        c2 = jnp.minimum(c0 + 2, CHUNKS_PER_TILE - 1)

        cp_a1 = pltpu.async_copy(tab.at[sidx.at[c1]], ra1, sem_a1)
        cp_b1 = pltpu.async_copy(tab.at[didx.at[c1]], rb1, sem_b1)

        pltpu.make_async_copy(tab.at[sidx.at[c0]], ra0, sem_a0).wait()
        pltpu.make_async_copy(tab.at[didx.at[c0]], rb0, sem_b0).wait()
        pltpu.make_async_copy(msg0, accum.at[didx.at[c0]], sem_s0).wait()
        _relu_unpack(ra0, rb0, msg0)
        pltpu.async_copy(tab.at[sidx.at[c2]], ra0, sem_a0)
        pltpu.async_copy(tab.at[didx.at[c2]], rb0, sem_b0)
        pltpu.async_copy(msg0, accum.at[didx.at[c0]], sem_s0, add=True)

        cp_a1.wait()
        cp_b1.wait()
        pltpu.make_async_copy(msg1, accum.at[didx.at[c1]], sem_s1).wait()
        _relu_unpack(ra1, rb1, msg1)
        pltpu.async_copy(msg1, accum.at[didx.at[c1]], sem_s1, add=True)

    pltpu.make_async_copy(
        tab.at[sidx.at[CHUNKS_PER_TILE - 1]], ra0, sem_a0).wait()
    pltpu.make_async_copy(
        tab.at[didx.at[CHUNKS_PER_TILE - 1]], rb0, sem_b0).wait()
    pltpu.make_async_copy(msg0, accum.at[didx.at[0]], sem_s0).wait()
    pltpu.make_async_copy(msg1, accum.at[didx.at[0]], sem_s1).wait()

    plsc.subcore_barrier()
    @pl.when(s < NS - 1)
    def _full():
        pltpu.sync_copy(
            accum.at[pl.ds(row0, ROWS_PER_TILE)],
            out_hbm.at[pl.ds(row0, ROWS_PER_TILE), c],
        )

    @pl.when(s == NS - 1)
    def _tail():
        pltpu.sync_copy(
            accum.at[pl.ds(row0, N_NODES - (NS - 1) * ROWS_PER_TILE)],
            out_hbm.at[pl.ds(row0, N_NODES - (NS - 1) * ROWS_PER_TILE), c],
        )


def kernel(x, edge_index, W, b):
    perm = jnp.asarray(_PERM)
    wp = W[:, perm]
    w1 = wp[:D]
    w2 = wp[D:]
    b2d = b[perm].reshape(1, D)
    table = _precompute_table(x, w1, w2, b2d).reshape(NC, 2 * N_NODES, H)
    srcg = edge_index[0].reshape(NS, CHUNKS_PER_TILE, CHUNK) + N_NODES
    dstg = edge_index[1].reshape(NS, CHUNKS_PER_TILE, CHUNK)
    out = _sc_edges(table, srcg, dstg)
    return out.reshape(N_NODES, D)


# speedup vs baseline: 1.0031x; 1.0031x over previous
import functools

import jax
import jax.numpy as jnp
import numpy as np
from jax import lax
from jax.experimental import pallas as pl
from jax.experimental.pallas import tpu as pltpu
from jax.experimental.pallas import tpu_sc as plsc

N_NODES = 10000
N_EDGES = 320000
D = 128
H = D // 2

NC = 2
NS = 16

CHUNK = 80
CHUNKS_PER_TILE = N_EDGES // (NS * CHUNK)

NP = 10240
ROWS_PER_TILE = NP // NS

BM = 400

_PERM = np.empty(D, dtype=np.int32)
for _h in range(4):
    _base = 32 * _h
    for _k in range(16):
        _PERM[_base + 2 * _k] = _base + _k
        _PERM[_base + 2 * _k + 1] = _base + 16 + _k


def _mm_body(x_ref, w1_ref, w2_ref, b_ref, t_ref):
    xb = x_ref[...]
    m1 = jnp.dot(xb, w1_ref[...], preferred_element_type=jnp.float32) + b_ref[...]
    m2 = jnp.dot(xb, w2_ref[...], preferred_element_type=jnp.float32)
    t_ref[0, 0] = m2[:, :H].astype(jnp.bfloat16)
    t_ref[0, 1] = m1[:, :H].astype(jnp.bfloat16)
    t_ref[1, 0] = m2[:, H:].astype(jnp.bfloat16)
    t_ref[1, 1] = m1[:, H:].astype(jnp.bfloat16)


def _precompute_table(x, w1, w2, b2d):
    return pl.pallas_call(
        _mm_body,
        grid=(N_NODES // BM,),
        in_specs=[
            pl.BlockSpec((BM, D), lambda i: (i, 0)),
            pl.BlockSpec((D, D), lambda i: (0, 0)),
            pl.BlockSpec((D, D), lambda i: (0, 0)),
            pl.BlockSpec((1, D), lambda i: (0, 0)),
        ],
        out_specs=pl.BlockSpec((NC, 2, BM, H), lambda i: (0, 0, i, 0)),
        out_shape=jax.ShapeDtypeStruct((NC, 2, N_NODES, H), jnp.bfloat16),
    )(x, w1, w2, b2d)


@functools.partial(
    pl.kernel,
    out_type=jax.ShapeDtypeStruct((N_NODES, NC, H), jnp.float32),
    mesh=plsc.VectorSubcoreMesh(core_axis_name="c", subcore_axis_name="s"),
    scratch_types=[
        pltpu.VMEM((CHUNKS_PER_TILE, CHUNK), jnp.int32),
        pltpu.VMEM((CHUNKS_PER_TILE, CHUNK), jnp.int32),
        pltpu.VMEM((CHUNK, H), jnp.bfloat16),
        pltpu.VMEM((CHUNK, H), jnp.bfloat16),
        pltpu.VMEM((CHUNK, H), jnp.bfloat16),
        pltpu.VMEM((CHUNK, H), jnp.bfloat16),
        pltpu.VMEM((CHUNK, H), jnp.float32),
        pltpu.VMEM((CHUNK, H), jnp.float32),
        pltpu.VMEM_SHARED((NP, H), jnp.float32),
        pltpu.SemaphoreType.DMA,
        pltpu.SemaphoreType.DMA,
        pltpu.SemaphoreType.DMA,
        pltpu.SemaphoreType.DMA,
        pltpu.SemaphoreType.DMA,
        pltpu.SemaphoreType.DMA,
    ],
    compiler_params=pltpu.CompilerParams(
        use_tc_tiling_on_sc=False, needs_layout_passes=False
    ),
)
def _sc_edges(t_hbm, srcg_hbm, dstg_hbm, out_hbm,
              sidx, didx, ra0, rb0, ra1, rb1, msg0, msg1, accum,
              sem_a0, sem_b0, sem_a1, sem_b1, sem_s0, sem_s1):
    c = lax.axis_index("c")
    s = lax.axis_index("s")
    tab = t_hbm.at[c]

    cp_si = pltpu.async_copy(srcg_hbm.at[s], sidx, sem_a0)
    cp_di = pltpu.async_copy(dstg_hbm.at[s], didx, sem_b0)

    zero = jnp.zeros((16,), jnp.float32)

    @pl.loop(0, CHUNK)
    def _zero_rows(e):
        for j in range(H // 16):
            msg0[e, pl.ds(j * 16, 16)] = zero
            msg1[e, pl.ds(j * 16, 16)] = zero

    row0 = s * ROWS_PER_TILE

    @pl.loop(0, ROWS_PER_TILE // CHUNK)
    def _zero_accum(k):
        pltpu.sync_copy(msg0, accum.at[pl.ds(row0 + k * CHUNK, CHUNK)])

    cp_si.wait()
    cp_di.wait()

    plsc.subcore_barrier()

    himask = jnp.full((16,), -65536, jnp.int32)

    def _relu_unpack(ra, rb, msg):
        @pl.loop(0, CHUNK, unroll=4)
        def _row(e):
            for j in range(H // 32):
                a32 = plsc.bitcast(ra[e, pl.ds(j * 32, 32)], jnp.int32)
                b32 = plsc.bitcast(rb[e, pl.ds(j * 32, 32)], jnp.int32)
                ae = plsc.bitcast(a32 << 16, jnp.float32)
                be = plsc.bitcast(b32 << 16, jnp.float32)
                ao = plsc.bitcast(a32 & himask, jnp.float32)
                bo = plsc.bitcast(b32 & himask, jnp.float32)
                msg[e, pl.ds(j * 32, 16)] = jnp.maximum(ae + be, 0.0)
                msg[e, pl.ds(j * 32 + 16, 16)] = jnp.maximum(ao + bo, 0.0)

    pltpu.async_copy(tab.at[sidx.at[0]], ra0, sem_a0)
    pltpu.async_copy(tab.at[didx.at[0]], rb0, sem_b0)
    pltpu.async_copy(msg0, accum.at[didx.at[0]], sem_s0, add=True)
    pltpu.async_copy(msg1, accum.at[didx.at[0]], sem_s1, add=True)

    @pl.loop(0, CHUNKS_PER_TILE // 2)
    def _pair(p):
        c0 = 2 * p
        c1 = c0 + 1
        c2 = jnp.minimum(c0 + 2, CHUNKS_PER_TILE - 1)

        cp_a1 = pltpu.async_copy(tab.at[sidx.at[c1]], ra1, sem_a1)
        cp_b1 = pltpu.async_copy(tab.at[didx.at[c1]], rb1, sem_b1)

        pltpu.make_async_copy(tab.at[sidx.at[c0]], ra0, sem_a0).wait()
        pltpu.make_async_copy(tab.at[didx.at[c0]], rb0, sem_b0).wait()
        pltpu.make_async_copy(msg0, accum.at[didx.at[c0]], sem_s0).wait()
        _relu_unpack(ra0, rb0, msg0)
        pltpu.async_copy(tab.at[sidx.at[c2]], ra0, sem_a0)
        pltpu.async_copy(tab.at[didx.at[c2]], rb0, sem_b0)
        pltpu.async_copy(msg0, accum.at[didx.at[c0]], sem_s0, add=True)

        cp_a1.wait()
        cp_b1.wait()
        pltpu.make_async_copy(msg1, accum.at[didx.at[c1]], sem_s1).wait()
        _relu_unpack(ra1, rb1, msg1)
        pltpu.async_copy(msg1, accum.at[didx.at[c1]], sem_s1, add=True)

    pltpu.make_async_copy(
        tab.at[sidx.at[CHUNKS_PER_TILE - 1]], ra0, sem_a0).wait()
    pltpu.make_async_copy(
        tab.at[didx.at[CHUNKS_PER_TILE - 1]], rb0, sem_b0).wait()
    pltpu.make_async_copy(msg0, accum.at[didx.at[0]], sem_s0).wait()
    pltpu.make_async_copy(msg1, accum.at[didx.at[0]], sem_s1).wait()

    plsc.subcore_barrier()
    @pl.when(s < NS - 1)
    def _full():
        pltpu.sync_copy(
            accum.at[pl.ds(row0, ROWS_PER_TILE)],
            out_hbm.at[pl.ds(row0, ROWS_PER_TILE), c],
        )

    @pl.when(s == NS - 1)
    def _tail():
        pltpu.sync_copy(
            accum.at[pl.ds(row0, N_NODES - (NS - 1) * ROWS_PER_TILE)],
            out_hbm.at[pl.ds(row0, N_NODES - (NS - 1) * ROWS_PER_TILE), c],
        )


def kernel(x, edge_index, W, b):
    perm = jnp.asarray(_PERM)
    wp = W[:, perm]
    w1 = wp[:D]
    w2 = wp[D:]
    b2d = b[perm].reshape(1, D)
    table = _precompute_table(x, w1, w2, b2d).reshape(NC, 2 * N_NODES, H)
    srcg = edge_index[0].reshape(NS, CHUNKS_PER_TILE, CHUNK) + N_NODES
    dstg = edge_index[1].reshape(NS, CHUNKS_PER_TILE, CHUNK)
    out = _sc_edges(table, srcg, dstg)
    return out.reshape(N_NODES, D)
